# Initial kernel scaffold; baseline (speedup 1.0000x reference)
import jax
import jax.numpy as jnp
from jax import lax
from jax.experimental import pallas as pl
from jax.experimental.pallas import tpu as pltpu
from jax.experimental.pallas import tpu_sc as plsc

_D = 128
_N = 10000
_DEG = 32
_P = 10240
_BLK = 512
_GRID = _P // _BLK
_NW = 32
_NPT = _P // _NW
_CN = 4
_CE = _CN * _DEG
_NCH = _NPT // _CN
_IDXROWS = _P * _DEG // _CE
_ROWS_PT = _NCH


def _embed_body(fa_ref, fb_ref, *refs):
    params = [refs[4 * r: 4 * r + 4] for r in range(4)]
    embs = refs[16:20]
    ssrcs = refs[20:24]
    sdsts = refs[24:28]
    fa = fa_ref[...]
    fb = fb_ref[...]
    dn = (((1,), (1,)), ((), ()))
    for r in range(4):
        W, b, lo, hi = params[r]
        fe = fb if r < 2 else fa
        fs = fa if r < 2 else fb
        emb = jnp.dot(fe, W[...], preferred_element_type=jnp.float32) + b[...]
        embs[r][...] = emb
        sd = lax.dot_general(hi[...], emb, dn, preferred_element_type=jnp.float32)
        sdsts[r][...] = sd.reshape(1, 1, _BLK)
        ss = lax.dot_general(lo[...], fs, dn, preferred_element_type=jnp.float32)
        ssrcs[r][...] = ss.reshape(1, 1, _BLK)


def _embed_call(fa, fb, params_flat):
    full = lambda i: (0, 0)
    row = lambda i: (i, 0)
    srow = lambda i: (i, 0, 0)
    in_specs = [pl.BlockSpec((_BLK, _D), row), pl.BlockSpec((_BLK, _D), row)]
    for _ in range(4):
        in_specs += [
            pl.BlockSpec((_D, _D), full),
            pl.BlockSpec((1, _D), full),
            pl.BlockSpec((1, _D), full),
            pl.BlockSpec((1, _D), full),
        ]
    out_specs = ([pl.BlockSpec((_BLK, _D), row)] * 4
                 + [pl.BlockSpec((1, 1, _BLK), srow)] * 8)
    out_shape = ([jax.ShapeDtypeStruct((_P, _D), jnp.float32)] * 4
                 + [jax.ShapeDtypeStruct((_GRID, 1, _BLK), jnp.float32)] * 8)
    return pl.pallas_call(
        _embed_body,
        grid=(_GRID,),
        in_specs=in_specs,
        out_specs=out_specs,
        out_shape=out_shape,
    )(fa, fb, *params_flat)


def _rel_agg(eh, dh, sh, dsh, oh, nbase, rbase,
             idx_v, sdst_v, ssrc_v, out_v, rows_v, w_v, sem):
    pltpu.sync_copy(dh.at[pl.ds(rbase, _ROWS_PT)], idx_v)
    pltpu.sync_copy(dsh, sdst_v)
    pltpu.sync_copy(sh.at[pl.ds(nbase, _NPT)], ssrc_v)

    def chunk(c, carry):
        pltpu.async_copy(eh.at[idx_v.at[c]], rows_v, sem).wait()
        for n in range(_CN):
            node = c * _CN + n
            eb = n * _DEG
            ssv = ssrc_v[node]
            wvecs = []
            tot = jnp.zeros((16,), jnp.float32)
            for k in range(_DEG // 16):
                ii = idx_v[c, pl.ds(eb + k * 16, 16)]
                sdv = plsc.load_gather(sdst_v, [ii])
                x = ssv + sdv
                w = jnp.exp(jnp.where(x > 0.0, x, 0.1 * (jnp.exp(x) - 1.0)))
                wvecs.append(w)
                tot = tot + w
            rs = jnp.sum(tot)
            inv = 1.0 / jnp.where(rs == 0.0, 1.0, rs)
            for k in range(_DEG // 16):
                w_v[pl.ds(k * 16, 16)] = wvecs[k] * inv

            def edge(j, acc):
                wj = plsc.load_gather(w_v, [jnp.full((16,), j, jnp.int32)])
                return tuple(acc[v] + wj * rows_v[eb + j, pl.ds(v * 16, 16)]
                             for v in range(_D // 16))

            acc0 = tuple(jnp.zeros((16,), jnp.float32) for _ in range(_D // 16))
            acc = lax.fori_loop(0, _DEG, edge, acc0, unroll=4)
            for v in range(_D // 16):
                out_v[node, pl.ds(v * 16, 16)] = acc[v]
        return carry

    lax.fori_loop(0, _NCH, chunk, 0)
    pltpu.sync_copy(out_v, oh.at[pl.ds(nbase, _NPT)])


def _agg_body(e0, e1, e2, e3, d0, d1, d2, d3,
              ss0, ss1, ss2, ss3, sd0, sd1, sd2, sd3,
              o0, o1, o2, o3,
              idx_v, sdst_v, ssrc_v, out_v, rows_v, w_v, sem):
    wid = lax.axis_index("s") * 2 + lax.axis_index("c")
    nbase = wid * _NPT
    rbase = wid * _ROWS_PT
    for eh, dh, sh, dsh, oh in ((e0, d0, ss0, sd0, o0),
                                (e1, d1, ss1, sd1, o1),
                                (e2, d2, ss2, sd2, o2),
                                (e3, d3, ss3, sd3, o3)):
        _rel_agg(eh, dh, sh, dsh, oh, nbase, rbase,
                 idx_v, sdst_v, ssrc_v, out_v, rows_v, w_v, sem)


def _agg_sc(embs, dsts, ssrcs, sdsts):
    mesh = plsc.VectorSubcoreMesh(core_axis_name="c", subcore_axis_name="s")
    call = pl.kernel(
        _agg_body,
        mesh=mesh,
        out_type=[jax.ShapeDtypeStruct((_P, _D), jnp.float32)] * 4,
        scratch_types=[
            pltpu.VMEM((_ROWS_PT, _CE), jnp.int32),
            pltpu.VMEM((_P,), jnp.float32),
            pltpu.VMEM((_NPT,), jnp.float32),
            pltpu.VMEM((_NPT, _D), jnp.float32),
            pltpu.VMEM((_CE, _D), jnp.float32),
            pltpu.VMEM((_DEG,), jnp.float32),
            pltpu.SemaphoreType.DMA,
        ],
    )
    return call(*embs, *dsts, *ssrcs, *sdsts)


def _mlp_body(f_ref, mp_ref, mn_ref, W1_ref, b1_ref, pw_ref, W2_ref, b2_ref, o_ref):
    W1 = W1_ref[...]
    h = (jnp.dot(f_ref[...], W1[0:_D], preferred_element_type=jnp.float32)
         + jnp.dot(mp_ref[...], W1[_D:2 * _D], preferred_element_type=jnp.float32)
         + jnp.dot(mn_ref[...], W1[2 * _D:3 * _D], preferred_element_type=jnp.float32)
         + b1_ref[...])
    h = jnp.where(h >= 0.0, h, pw_ref[...] * h)
    o_ref[...] = jnp.dot(h, W2_ref[...], preferred_element_type=jnp.float32) + b2_ref[...]


def _mlp_call(f, mp, mn, W1, b1, pw_row, W2, b2):
    full = lambda i: (0, 0)
    row = lambda i: (i, 0)
    return pl.pallas_call(
        _mlp_body,
        grid=(_GRID,),
        in_specs=[
            pl.BlockSpec((_BLK, _D), row),
            pl.BlockSpec((_BLK, _D), row),
            pl.BlockSpec((_BLK, _D), row),
            pl.BlockSpec((3 * _D, 2 * _D), full),
            pl.BlockSpec((1, 2 * _D), full),
            pl.BlockSpec((1, 2 * _D), full),
            pl.BlockSpec((2 * _D, _D), full),
            pl.BlockSpec((1, _D), full),
        ],
        out_specs=pl.BlockSpec((_BLK, _D), row),
        out_shape=jax.ShapeDtypeStruct((_N, _D), jnp.float32),
    )(f, mp, mn, W1, b1, pw_row, W2, b2)


def kernel(feature_a, feature_b, edge_ab_pos, edge_ab_neg, edge_ba_pos, edge_ba_neg,
           W_abp, b_abp, a_abp, W_abn, b_abn, a_abn,
           W_bap, b_bap, a_bap, W_ban, b_ban, a_ban,
           W1, b1, prelu_w, W2, b2):
    fa = jnp.pad(feature_a, ((0, _P - _N), (0, 0)))
    fb = jnp.pad(feature_b, ((0, _P - _N), (0, 0)))

    def prep_dst(e):
        return jnp.pad(e[1], (0, _P * _DEG - e.shape[1])).reshape(_IDXROWS, _CE)

    dsts = [prep_dst(edge_ab_pos), prep_dst(edge_ab_neg),
            prep_dst(edge_ba_pos), prep_dst(edge_ba_neg)]

    params_flat = []
    for W, b, a in ((W_abp, b_abp, a_abp), (W_abn, b_abn, a_abn),
                    (W_bap, b_bap, a_bap), (W_ban, b_ban, a_ban)):
        params_flat += [W, b.reshape(1, _D),
                        a[:_D, 0].reshape(1, _D), a[_D:, 0].reshape(1, _D)]

    res = _embed_call(fa, fb, params_flat)
    embs = res[0:4]
    ssrcs = [x.reshape(_P) for x in res[4:8]]
    sdsts = [x.reshape(_P) for x in res[8:12]]

    m0, m1, m2, m3 = _agg_sc(embs, dsts, ssrcs, sdsts)

    b1r = b1.reshape(1, 2 * _D)
    b2r = b2.reshape(1, _D)
    pw_row = jnp.broadcast_to(prelu_w.reshape(1, 1), (1, 2 * _D))
    out_a = _mlp_call(fa, m0, m1, W1, b1r, pw_row, W2, b2r)
    out_b = _mlp_call(fb, m2, m3, W1, b1r, pw_row, W2, b2r)
    return (out_a, out_b)


# speedup vs baseline: 4.4442x; 4.4442x over previous
import jax
import jax.numpy as jnp
from jax import lax
from jax.experimental import pallas as pl
from jax.experimental.pallas import tpu as pltpu
from jax.experimental.pallas import tpu_sc as plsc

_D = 128
_N = 10000
_DEG = 32
_P = 10240
_BLK = 512
_GRID = _P // _BLK
_NW = 32
_NPT = _P // _NW
_CN = 4
_CE = _CN * _DEG
_NCH = _NPT // _CN
_IDXROWS = _P * _DEG // _CE
_ROWS_PT = _NCH


def _embed_body(fa_ref, fb_ref, *refs):
    params = [refs[4 * r: 4 * r + 4] for r in range(4)]
    embs = refs[16:20]
    ssrcs = refs[20:24]
    sdsts = refs[24:28]
    fa = fa_ref[...]
    fb = fb_ref[...]
    dn = (((1,), (1,)), ((), ()))
    for r in range(4):
        W, b, lo, hi = params[r]
        fe = fb if r < 2 else fa
        fs = fa if r < 2 else fb
        emb = jnp.dot(fe, W[...], preferred_element_type=jnp.float32) + b[...]
        embs[r][...] = emb
        sd = lax.dot_general(hi[...], emb, dn, preferred_element_type=jnp.float32)
        sdsts[r][...] = sd.reshape(1, 1, _BLK)
        ss = lax.dot_general(lo[...], fs, dn, preferred_element_type=jnp.float32)
        ssrcs[r][...] = ss.reshape(1, 1, _BLK)


def _embed_call(fa, fb, params_flat):
    full = lambda i: (0, 0)
    row = lambda i: (i, 0)
    srow = lambda i: (i, 0, 0)
    in_specs = [pl.BlockSpec((_BLK, _D), row), pl.BlockSpec((_BLK, _D), row)]
    for _ in range(4):
        in_specs += [
            pl.BlockSpec((_D, _D), full),
            pl.BlockSpec((1, _D), full),
            pl.BlockSpec((1, _D), full),
            pl.BlockSpec((1, _D), full),
        ]
    out_specs = ([pl.BlockSpec((_BLK, _D), row)] * 4
                 + [pl.BlockSpec((1, 1, _BLK), srow)] * 8)
    out_shape = ([jax.ShapeDtypeStruct((_P, _D), jnp.float32)] * 4
                 + [jax.ShapeDtypeStruct((_GRID, 1, _BLK), jnp.float32)] * 8)
    return pl.pallas_call(
        _embed_body,
        grid=(_GRID,),
        in_specs=in_specs,
        out_specs=out_specs,
        out_shape=out_shape,
    )(fa, fb, *params_flat)


def _rel_agg(eh, dh, sh, dsh, oh, nbase, rbase,
             idx_v, sdst_v, ssrc_v, out_v, rows_v, w_v, sem):
    pltpu.sync_copy(dh.at[pl.ds(rbase, _ROWS_PT)], idx_v)
    pltpu.sync_copy(dsh, sdst_v)
    pltpu.sync_copy(sh.at[pl.ds(nbase, _NPT)], ssrc_v.at[pl.ds(0, _NPT)])

    def chunk(c, carry):
        pltpu.async_copy(eh.at[idx_v.at[c]], rows_v, sem).wait()
        sv = ssrc_v[pl.ds(c * _CN, 16)]
        for n in range(_CN):
            node = c * _CN + n
            eb = n * _DEG
            ssv = sv[n]
            wvecs = []
            tot = jnp.zeros((16,), jnp.float32)
            for k in range(_DEG // 16):
                ii = idx_v[c, pl.ds(eb + k * 16, 16)]
                sdv = plsc.load_gather(sdst_v, [ii])
                x = ssv + sdv
                w = jnp.exp(jnp.where(x > 0.0, x, 0.1 * (jnp.exp(x) - 1.0)))
                wvecs.append(w)
                tot = tot + w
            rsv = jnp.full((16,), jnp.sum(tot), jnp.float32)
            invv = 1.0 / jnp.where(rsv == 0.0, 1.0, rsv)
            for k in range(_DEG // 16):
                w_v[pl.ds(k * 16, 16)] = wvecs[k] * invv

            def edge(j, acc):
                wj = plsc.load_gather(w_v, [jnp.full((16,), j, jnp.int32)])
                return tuple(acc[v] + wj * rows_v[eb + j, pl.ds(v * 16, 16)]
                             for v in range(_D // 16))

            acc0 = tuple(jnp.zeros((16,), jnp.float32) for _ in range(_D // 16))
            acc = lax.fori_loop(0, _DEG, edge, acc0, unroll=4)
            for v in range(_D // 16):
                out_v[node, pl.ds(v * 16, 16)] = acc[v]
        return carry

    lax.fori_loop(0, _NCH, chunk, 0)
    pltpu.sync_copy(out_v, oh.at[pl.ds(nbase, _NPT)])


def _agg_body(e0, e1, e2, e3, d0, d1, d2, d3,
              ss0, ss1, ss2, ss3, sd0, sd1, sd2, sd3,
              o0, o1, o2, o3,
              idx_v, sdst_v, ssrc_v, out_v, rows_v, w_v, sem):
    wid = lax.axis_index("s") * 2 + lax.axis_index("c")
    nbase = wid * _NPT
    rbase = wid * _ROWS_PT
    for eh, dh, sh, dsh, oh in ((e0, d0, ss0, sd0, o0),
                                (e1, d1, ss1, sd1, o1),
                                (e2, d2, ss2, sd2, o2),
                                (e3, d3, ss3, sd3, o3)):
        _rel_agg(eh, dh, sh, dsh, oh, nbase, rbase,
                 idx_v, sdst_v, ssrc_v, out_v, rows_v, w_v, sem)


def _agg_sc(embs, dsts, ssrcs, sdsts):
    mesh = plsc.VectorSubcoreMesh(core_axis_name="c", subcore_axis_name="s")
    call = pl.kernel(
        _agg_body,
        mesh=mesh,
        compiler_params=pltpu.CompilerParams(use_tc_tiling_on_sc=False,
                                             needs_layout_passes=False),
        out_type=[jax.ShapeDtypeStruct((_P, _D), jnp.float32)] * 4,
        scratch_types=[
            pltpu.VMEM((_ROWS_PT, _CE), jnp.int32),
            pltpu.VMEM((_P,), jnp.float32),
            pltpu.VMEM((_NPT + 16,), jnp.float32),
            pltpu.VMEM((_NPT, _D), jnp.float32),
            pltpu.VMEM((_CE, _D), jnp.float32),
            pltpu.VMEM((_DEG,), jnp.float32),
            pltpu.SemaphoreType.DMA,
        ],
    )
    return call(*embs, *dsts, *ssrcs, *sdsts)


def _mlp_body(f_ref, mp_ref, mn_ref, W1_ref, b1_ref, pw_ref, W2_ref, b2_ref, o_ref):
    W1 = W1_ref[...]
    h = (jnp.dot(f_ref[...], W1[0:_D], preferred_element_type=jnp.float32)
         + jnp.dot(mp_ref[...], W1[_D:2 * _D], preferred_element_type=jnp.float32)
         + jnp.dot(mn_ref[...], W1[2 * _D:3 * _D], preferred_element_type=jnp.float32)
         + b1_ref[...])
    h = jnp.where(h >= 0.0, h, pw_ref[...] * h)
    o_ref[...] = jnp.dot(h, W2_ref[...], preferred_element_type=jnp.float32) + b2_ref[...]


def _mlp_call(f, mp, mn, W1, b1, pw_row, W2, b2):
    full = lambda i: (0, 0)
    row = lambda i: (i, 0)
    return pl.pallas_call(
        _mlp_body,
        grid=(_GRID,),
        in_specs=[
            pl.BlockSpec((_BLK, _D), row),
            pl.BlockSpec((_BLK, _D), row),
            pl.BlockSpec((_BLK, _D), row),
            pl.BlockSpec((3 * _D, 2 * _D), full),
            pl.BlockSpec((1, 2 * _D), full),
            pl.BlockSpec((1, 2 * _D), full),
            pl.BlockSpec((2 * _D, _D), full),
            pl.BlockSpec((1, _D), full),
        ],
        out_specs=pl.BlockSpec((_BLK, _D), row),
        out_shape=jax.ShapeDtypeStruct((_N, _D), jnp.float32),
    )(f, mp, mn, W1, b1, pw_row, W2, b2)


def kernel(feature_a, feature_b, edge_ab_pos, edge_ab_neg, edge_ba_pos, edge_ba_neg,
           W_abp, b_abp, a_abp, W_abn, b_abn, a_abn,
           W_bap, b_bap, a_bap, W_ban, b_ban, a_ban,
           W1, b1, prelu_w, W2, b2):
    fa = jnp.pad(feature_a, ((0, _P - _N), (0, 0)))
    fb = jnp.pad(feature_b, ((0, _P - _N), (0, 0)))

    def prep_dst(e):
        return jnp.pad(e[1], (0, _P * _DEG - e.shape[1])).reshape(_IDXROWS, _CE)

    dsts = [prep_dst(edge_ab_pos), prep_dst(edge_ab_neg),
            prep_dst(edge_ba_pos), prep_dst(edge_ba_neg)]

    params_flat = []
    for W, b, a in ((W_abp, b_abp, a_abp), (W_abn, b_abn, a_abn),
                    (W_bap, b_bap, a_bap), (W_ban, b_ban, a_ban)):
        params_flat += [W, b.reshape(1, _D),
                        a[:_D, 0].reshape(1, _D), a[_D:, 0].reshape(1, _D)]

    res = _embed_call(fa, fb, params_flat)
    embs = res[0:4]
    ssrcs = [x.reshape(_P) for x in res[4:8]]
    sdsts = [x.reshape(_P) for x in res[8:12]]

    m0, m1, m2, m3 = _agg_sc(embs, dsts, ssrcs, sdsts)

    b1r = b1.reshape(1, 2 * _D)
    b2r = b2.reshape(1, _D)
    pw_row = jnp.broadcast_to(prelu_w.reshape(1, 1), (1, 2 * _D))
    out_a = _mlp_call(fa, m0, m1, W1, b1r, pw_row, W2, b2r)
    out_b = _mlp_call(fb, m2, m3, W1, b1r, pw_row, W2, b2r)
    return (out_a, out_b)


# speedup vs baseline: 5.0576x; 1.1380x over previous
import jax
import jax.numpy as jnp
from jax import lax
from jax.experimental import pallas as pl
from jax.experimental.pallas import tpu as pltpu
from jax.experimental.pallas import tpu_sc as plsc

_D = 128
_N = 10000
_DEG = 32
_P = 10240
_BLK = 512
_GRID = _P // _BLK
_NW = 32
_NPT = _P // _NW
_CN = 4
_CE = _CN * _DEG
_NCH = _NPT // _CN
_IDXROWS = _P * _DEG // _CE
_ROWS_PT = _NCH


def _embed_body(fa_ref, fb_ref, *refs):
    params = [refs[4 * r: 4 * r + 4] for r in range(4)]
    embs = refs[16:20]
    ssrcs = refs[20:24]
    sdsts = refs[24:28]
    fa = fa_ref[...]
    fb = fb_ref[...]
    dn = (((1,), (1,)), ((), ()))
    for r in range(4):
        W, b, lo, hi = params[r]
        fe = fb if r < 2 else fa
        fs = fa if r < 2 else fb
        emb = jnp.dot(fe, W[...], preferred_element_type=jnp.float32) + b[...]
        embs[r][...] = emb
        sd = lax.dot_general(hi[...], emb, dn, preferred_element_type=jnp.float32)
        sdsts[r][...] = sd.reshape(1, 1, _BLK)
        ss = lax.dot_general(lo[...], fs, dn, preferred_element_type=jnp.float32)
        ssrcs[r][...] = ss.reshape(1, 1, _BLK)


def _embed_call(fa, fb, params_flat):
    full = lambda i: (0, 0)
    row = lambda i: (i, 0)
    srow = lambda i: (i, 0, 0)
    in_specs = [pl.BlockSpec((_BLK, _D), row), pl.BlockSpec((_BLK, _D), row)]
    for _ in range(4):
        in_specs += [
            pl.BlockSpec((_D, _D), full),
            pl.BlockSpec((1, _D), full),
            pl.BlockSpec((1, _D), full),
            pl.BlockSpec((1, _D), full),
        ]
    out_specs = ([pl.BlockSpec((_BLK, _D), row)] * 4
                 + [pl.BlockSpec((1, 1, _BLK), srow)] * 8)
    out_shape = ([jax.ShapeDtypeStruct((_P, _D), jnp.float32)] * 4
                 + [jax.ShapeDtypeStruct((_GRID, 1, _BLK), jnp.float32)] * 8)
    return pl.pallas_call(
        _embed_body,
        grid=(_GRID,),
        in_specs=in_specs,
        out_specs=out_specs,
        out_shape=out_shape,
    )(fa, fb, *params_flat)


def _rel_agg(eh, dh, sh, dsh, oh, nbase, rbase,
             idx_v, sdst_v, ssrc_v, out_v, rows0, rows1, w_v, sem0, sem1):
    pltpu.sync_copy(dh.at[pl.ds(rbase, _ROWS_PT)], idx_v.at[pl.ds(0, _ROWS_PT)])
    pltpu.sync_copy(dh.at[pl.ds(rbase, 2)], idx_v.at[pl.ds(_ROWS_PT, 2)])
    pltpu.sync_copy(dsh, sdst_v)
    pltpu.sync_copy(sh.at[pl.ds(nbase, _NPT)], ssrc_v.at[pl.ds(0, _NPT)])

    def compute(c, rows_v):
        sv = ssrc_v[pl.ds(c * _CN, 16)]
        for n in range(_CN):
            node = c * _CN + n
            eb = n * _DEG
            ssv = sv[n]
            wvecs = []
            tot = jnp.zeros((16,), jnp.float32)
            for k in range(_DEG // 16):
                ii = idx_v[c, pl.ds(eb + k * 16, 16)]
                sdv = plsc.load_gather(sdst_v, [ii])
                x = ssv + sdv
                w = jnp.exp(jnp.where(x > 0.0, x, 0.1 * (jnp.exp(x) - 1.0)))
                wvecs.append(w)
                tot = tot + w
            rsv = jnp.full((16,), jnp.sum(tot), jnp.float32)
            invv = 1.0 / jnp.where(rsv == 0.0, 1.0, rsv)
            for k in range(_DEG // 16):
                w_v[pl.ds(k * 16, 16)] = wvecs[k] * invv

            def edge(j, acc):
                wj = plsc.load_gather(w_v, [jnp.full((16,), j, jnp.int32)])
                return tuple(acc[v] + wj * rows_v[eb + j, pl.ds(v * 16, 16)]
                             for v in range(_D // 16))

            acc0 = tuple(jnp.zeros((16,), jnp.float32) for _ in range(_D // 16))
            acc = lax.fori_loop(0, _DEG, edge, acc0, unroll=4)
            for v in range(_D // 16):
                out_v[node, pl.ds(v * 16, 16)] = acc[v]

    pltpu.async_copy(eh.at[idx_v.at[0]], rows0, sem0)
    pltpu.async_copy(eh.at[idx_v.at[1]], rows1, sem1)

    def pair(g, carry):
        c = 2 * g
        pltpu.make_async_copy(eh.at[idx_v.at[c]], rows0, sem0).wait()
        compute(c, rows0)
        pltpu.async_copy(eh.at[idx_v.at[c + 2]], rows0, sem0)
        pltpu.make_async_copy(eh.at[idx_v.at[c + 1]], rows1, sem1).wait()
        compute(c + 1, rows1)
        pltpu.async_copy(eh.at[idx_v.at[c + 3]], rows1, sem1)
        return carry

    lax.fori_loop(0, _NCH // 2, pair, 0)
    pltpu.make_async_copy(eh.at[idx_v.at[_ROWS_PT]], rows0, sem0).wait()
    pltpu.make_async_copy(eh.at[idx_v.at[_ROWS_PT + 1]], rows1, sem1).wait()
    pltpu.sync_copy(out_v, oh.at[pl.ds(nbase, _NPT)])


def _agg_body(e0, e1, e2, e3, d0, d1, d2, d3,
              ss0, ss1, ss2, ss3, sd0, sd1, sd2, sd3,
              o0, o1, o2, o3,
              idx_v, sdst_v, ssrc_v, out_v, rows0, rows1, w_v, sem0, sem1):
    wid = lax.axis_index("s") * 2 + lax.axis_index("c")
    nbase = wid * _NPT
    rbase = wid * _ROWS_PT
    for eh, dh, sh, dsh, oh in ((e0, d0, ss0, sd0, o0),
                                (e1, d1, ss1, sd1, o1),
                                (e2, d2, ss2, sd2, o2),
                                (e3, d3, ss3, sd3, o3)):
        _rel_agg(eh, dh, sh, dsh, oh, nbase, rbase,
                 idx_v, sdst_v, ssrc_v, out_v, rows0, rows1, w_v, sem0, sem1)


def _agg_sc(embs, dsts, ssrcs, sdsts):
    mesh = plsc.VectorSubcoreMesh(core_axis_name="c", subcore_axis_name="s")
    call = pl.kernel(
        _agg_body,
        mesh=mesh,
        compiler_params=pltpu.CompilerParams(use_tc_tiling_on_sc=False,
                                             needs_layout_passes=False),
        out_type=[jax.ShapeDtypeStruct((_P, _D), jnp.float32)] * 4,
        scratch_types=[
            pltpu.VMEM((_ROWS_PT + 2, _CE), jnp.int32),
            pltpu.VMEM((_P,), jnp.float32),
            pltpu.VMEM((_NPT + 16,), jnp.float32),
            pltpu.VMEM((_NPT, _D), jnp.float32),
            pltpu.VMEM((_CE, _D), jnp.float32),
            pltpu.VMEM((_CE, _D), jnp.float32),
            pltpu.VMEM((_DEG,), jnp.float32),
            pltpu.SemaphoreType.DMA,
            pltpu.SemaphoreType.DMA,
        ],
    )
    return call(*embs, *dsts, *ssrcs, *sdsts)


def _mlp_body(f_ref, mp_ref, mn_ref, W1_ref, b1_ref, pw_ref, W2_ref, b2_ref, o_ref):
    W1 = W1_ref[...]
    h = (jnp.dot(f_ref[...], W1[0:_D], preferred_element_type=jnp.float32)
         + jnp.dot(mp_ref[...], W1[_D:2 * _D], preferred_element_type=jnp.float32)
         + jnp.dot(mn_ref[...], W1[2 * _D:3 * _D], preferred_element_type=jnp.float32)
         + b1_ref[...])
    h = jnp.where(h >= 0.0, h, pw_ref[...] * h)
    o_ref[...] = jnp.dot(h, W2_ref[...], preferred_element_type=jnp.float32) + b2_ref[...]


def _mlp_call(f, mp, mn, W1, b1, pw_row, W2, b2):
    full = lambda i: (0, 0)
    row = lambda i: (i, 0)
    return pl.pallas_call(
        _mlp_body,
        grid=(_GRID,),
        in_specs=[
            pl.BlockSpec((_BLK, _D), row),
            pl.BlockSpec((_BLK, _D), row),
            pl.BlockSpec((_BLK, _D), row),
            pl.BlockSpec((3 * _D, 2 * _D), full),
            pl.BlockSpec((1, 2 * _D), full),
            pl.BlockSpec((1, 2 * _D), full),
            pl.BlockSpec((2 * _D, _D), full),
            pl.BlockSpec((1, _D), full),
        ],
        out_specs=pl.BlockSpec((_BLK, _D), row),
        out_shape=jax.ShapeDtypeStruct((_N, _D), jnp.float32),
    )(f, mp, mn, W1, b1, pw_row, W2, b2)


def kernel(feature_a, feature_b, edge_ab_pos, edge_ab_neg, edge_ba_pos, edge_ba_neg,
           W_abp, b_abp, a_abp, W_abn, b_abn, a_abn,
           W_bap, b_bap, a_bap, W_ban, b_ban, a_ban,
           W1, b1, prelu_w, W2, b2):
    fa = jnp.pad(feature_a, ((0, _P - _N), (0, 0)))
    fb = jnp.pad(feature_b, ((0, _P - _N), (0, 0)))

    def prep_dst(e):
        return jnp.pad(e[1], (0, _P * _DEG - e.shape[1])).reshape(_IDXROWS, _CE)

    dsts = [prep_dst(edge_ab_pos), prep_dst(edge_ab_neg),
            prep_dst(edge_ba_pos), prep_dst(edge_ba_neg)]

    params_flat = []
    for W, b, a in ((W_abp, b_abp, a_abp), (W_abn, b_abn, a_abn),
                    (W_bap, b_bap, a_bap), (W_ban, b_ban, a_ban)):
        params_flat += [W, b.reshape(1, _D),
                        a[:_D, 0].reshape(1, _D), a[_D:, 0].reshape(1, _D)]

    res = _embed_call(fa, fb, params_flat)
    embs = res[0:4]
    ssrcs = [x.reshape(_P) for x in res[4:8]]
    sdsts = [x.reshape(_P) for x in res[8:12]]

    m0, m1, m2, m3 = _agg_sc(embs, dsts, ssrcs, sdsts)

    b1r = b1.reshape(1, 2 * _D)
    b2r = b2.reshape(1, _D)
    pw_row = jnp.broadcast_to(prelu_w.reshape(1, 1), (1, 2 * _D))
    out_a = _mlp_call(fa, m0, m1, W1, b1r, pw_row, W2, b2r)
    out_b = _mlp_call(fb, m2, m3, W1, b1r, pw_row, W2, b2r)
    return (out_a, out_b)


# speedup vs baseline: 21.3943x; 4.2302x over previous
import jax
import jax.numpy as jnp
from jax import lax
from jax.experimental import pallas as pl
from jax.experimental.pallas import tpu as pltpu
from jax.experimental.pallas import tpu_sc as plsc

_D = 128
_N = 10000
_DEG = 32
_P = 10240
_BLK = 512
_GRID = _P // _BLK
_NW = 32
_NPT = _P // _NW
_CN = 4
_CE = _CN * _DEG
_NCH = _NPT // _CN
_IDXROWS = _P * _DEG // _CE
_ROWS_PT = _NCH


def _embed_body(fa_ref, fb_ref, *refs):
    params = [refs[4 * r: 4 * r + 4] for r in range(4)]
    embs = refs[16:20]
    ssrcs = refs[20:24]
    sdsts = refs[24:28]
    fa = fa_ref[...]
    fb = fb_ref[...]
    dn = (((1,), (1,)), ((), ()))
    for r in range(4):
        W, b, lo, hi = params[r]
        fe = fb if r < 2 else fa
        fs = fa if r < 2 else fb
        emb = jnp.dot(fe, W[...], preferred_element_type=jnp.float32) + b[...]
        embs[r][...] = emb.astype(jnp.bfloat16)
        sd = lax.dot_general(hi[...], emb, dn, preferred_element_type=jnp.float32)
        sdsts[r][...] = sd.reshape(_BLK)
        ss = lax.dot_general(lo[...], fs, dn, preferred_element_type=jnp.float32)
        ssrcs[r][...] = ss.reshape(_BLK)


def _embed_call(fa, fb, params_flat):
    full = lambda i: (0, 0)
    row = lambda i: (i, 0)
    srow = lambda i: (i, 0, 0)
    in_specs = [pl.BlockSpec((_BLK, _D), row), pl.BlockSpec((_BLK, _D), row)]
    for _ in range(4):
        in_specs += [
            pl.BlockSpec((_D, _D), full),
            pl.BlockSpec((1, _D), full),
            pl.BlockSpec((1, _D), full),
            pl.BlockSpec((1, _D), full),
        ]
    out_specs = ([pl.BlockSpec((_BLK, _D), row)] * 4
                 + [pl.BlockSpec((_BLK,), lambda i: (i,))] * 8)
    out_shape = ([jax.ShapeDtypeStruct((_P, _D), jnp.bfloat16)] * 4
                 + [jax.ShapeDtypeStruct((_P,), jnp.float32)] * 8)
    return pl.pallas_call(
        _embed_body,
        grid=(_GRID,),
        in_specs=in_specs,
        out_specs=out_specs,
        out_shape=out_shape,
    )(fa, fb, *params_flat)


def _rel_agg(eh, dh, sh, dsh, oh, nbase, ebase, sid,
             idx_v, sdst_v, ssrc_v, out_v, rows0, rows1, w_v, tbl_sp,
             sem0, sem1):
    @pl.when(sid == 0)
    def _():
        pltpu.sync_copy(eh, tbl_sp)
    pltpu.sync_copy(dh.at[1, pl.ds(ebase, _NPT * _DEG)],
                    idx_v.at[pl.ds(0, _NPT * _DEG)])
    pltpu.sync_copy(dh.at[1, pl.ds(ebase, 2 * _CE)],
                    idx_v.at[pl.ds(_NPT * _DEG, 2 * _CE)])
    pltpu.sync_copy(dsh, sdst_v)
    pltpu.sync_copy(sh.at[pl.ds(nbase, _NPT)], ssrc_v.at[pl.ds(0, _NPT)])
    plsc.subcore_barrier()

    def compute(c, rows_v):
        sv = ssrc_v[pl.ds(c * _CN, 16)]
        for n in range(_CN):
            node = c * _CN + n
            eb = n * _DEG
            ssv = sv[n]
            wvecs = []
            tot = jnp.zeros((16,), jnp.float32)
            for k in range(_DEG // 16):
                ii = idx_v[pl.ds(c * _CE + eb + k * 16, 16)]
                sdv = plsc.load_gather(sdst_v, [ii])
                x = ssv + sdv
                w = jnp.exp(jnp.where(x > 0.0, x, 0.1 * (jnp.exp(x) - 1.0)))
                wvecs.append(w)
                tot = tot + w
            rsv = jnp.full((16,), jnp.sum(tot), jnp.float32)
            invv = 1.0 / jnp.where(rsv == 0.0, 1.0, rsv)
            for k in range(_DEG // 16):
                w_v[pl.ds(k * 16, 16)] = wvecs[k] * invv

            def edge(j, acc):
                wj = plsc.load_gather(w_v, [jnp.full((16,), j, jnp.int32)])
                outs = []
                for v in range(_D // 32):
                    pk = rows_v[eb + j, pl.ds(v * 32, 32)]
                    pi = plsc.bitcast(pk, jnp.int32)
                    ev = plsc.bitcast(lax.shift_left(pi, 16), jnp.float32)
                    od = plsc.bitcast(jnp.bitwise_and(pi, jnp.int32(-65536)),
                                      jnp.float32)
                    outs.append(acc[2 * v] + wj * ev)
                    outs.append(acc[2 * v + 1] + wj * od)
                return tuple(outs)

            acc0 = tuple(jnp.zeros((16,), jnp.float32) for _ in range(_D // 16))
            acc = lax.fori_loop(0, _DEG, edge, acc0, unroll=4)
            for v in range(_D // 32):
                out_v[node, pl.ds(v * 32, 16)] = acc[2 * v]
                out_v[node, pl.ds(v * 32 + 16, 16)] = acc[2 * v + 1]

    def irow(c):
        return idx_v.at[pl.ds(c * _CE, _CE)]

    pltpu.async_copy(tbl_sp.at[irow(0)], rows0, sem0)
    pltpu.async_copy(tbl_sp.at[irow(1)], rows1, sem1)

    def pair(g, carry):
        c = 2 * g
        pltpu.make_async_copy(tbl_sp.at[irow(c)], rows0, sem0).wait()
        compute(c, rows0)
        pltpu.async_copy(tbl_sp.at[irow(c + 2)], rows0, sem0)
        pltpu.make_async_copy(tbl_sp.at[irow(c + 1)], rows1, sem1).wait()
        compute(c + 1, rows1)
        pltpu.async_copy(tbl_sp.at[irow(c + 3)], rows1, sem1)
        return carry

    lax.fori_loop(0, _NCH // 2, pair, 0)
    pltpu.make_async_copy(tbl_sp.at[irow(_NCH)], rows0, sem0).wait()
    pltpu.make_async_copy(tbl_sp.at[irow(_NCH + 1)], rows1, sem1).wait()
    pltpu.sync_copy(out_v, oh.at[pl.ds(nbase, _NPT)])
    plsc.subcore_barrier()


def _agg_body(e0, e1, e2, e3, d0, d1, d2, d3,
              ss0, ss1, ss2, ss3, sd0, sd1, sd2, sd3,
              o0, o1, o2, o3,
              idx_v, sdst_v, ssrc_v, out_v, rows0, rows1, w_v, tbl_sp,
              sem0, sem1):
    sid = lax.axis_index("s")
    wid = sid * 2 + lax.axis_index("c")
    nbase = jnp.minimum(wid * _NPT, _N - _NPT)
    ebase = nbase * _DEG
    for eh, dh, sh, dsh, oh in ((e0, d0, ss0, sd0, o0),
                                (e1, d1, ss1, sd1, o1),
                                (e2, d2, ss2, sd2, o2),
                                (e3, d3, ss3, sd3, o3)):
        _rel_agg(eh, dh, sh, dsh, oh, nbase, ebase, sid,
                 idx_v, sdst_v, ssrc_v, out_v, rows0, rows1, w_v, tbl_sp,
                 sem0, sem1)


def _agg_sc(embs, dsts, ssrcs, sdsts):
    mesh = plsc.VectorSubcoreMesh(core_axis_name="c", subcore_axis_name="s")
    call = pl.kernel(
        _agg_body,
        mesh=mesh,
        compiler_params=pltpu.CompilerParams(use_tc_tiling_on_sc=False,
                                             needs_layout_passes=False),
        out_type=[jax.ShapeDtypeStruct((_N, _D), jnp.float32)] * 4,
        scratch_types=[
            pltpu.VMEM((_NPT * _DEG + 2 * _CE,), jnp.int32),
            pltpu.VMEM((_P,), jnp.float32),
            pltpu.VMEM((_NPT + 16,), jnp.float32),
            pltpu.VMEM((_NPT, _D), jnp.float32),
            pltpu.VMEM((_CE, _D), jnp.bfloat16),
            pltpu.VMEM((_CE, _D), jnp.bfloat16),
            pltpu.VMEM((_DEG,), jnp.float32),
            pltpu.VMEM_SHARED((_P, _D), jnp.bfloat16),
            pltpu.SemaphoreType.DMA,
            pltpu.SemaphoreType.DMA,
        ],
    )
    return call(*embs, *dsts, *ssrcs, *sdsts)


def _mlp_body(f_ref, mp_ref, mn_ref, W1_ref, b1_ref, pw_ref, W2_ref, b2_ref, o_ref):
    W1 = W1_ref[...]
    h = (jnp.dot(f_ref[...], W1[0:_D], preferred_element_type=jnp.float32)
         + jnp.dot(mp_ref[...], W1[_D:2 * _D], preferred_element_type=jnp.float32)
         + jnp.dot(mn_ref[...], W1[2 * _D:3 * _D], preferred_element_type=jnp.float32)
         + b1_ref[...])
    h = jnp.where(h >= 0.0, h, pw_ref[...] * h)
    o_ref[...] = jnp.dot(h, W2_ref[...], preferred_element_type=jnp.float32) + b2_ref[...]


def _mlp_call(f, mp, mn, W1, b1, pw_row, W2, b2):
    full = lambda i: (0, 0)
    row = lambda i: (i, 0)
    return pl.pallas_call(
        _mlp_body,
        grid=(_GRID,),
        in_specs=[
            pl.BlockSpec((_BLK, _D), row),
            pl.BlockSpec((_BLK, _D), row),
            pl.BlockSpec((_BLK, _D), row),
            pl.BlockSpec((3 * _D, 2 * _D), full),
            pl.BlockSpec((1, 2 * _D), full),
            pl.BlockSpec((1, 2 * _D), full),
            pl.BlockSpec((2 * _D, _D), full),
            pl.BlockSpec((1, _D), full),
        ],
        out_specs=pl.BlockSpec((_BLK, _D), row),
        out_shape=jax.ShapeDtypeStruct((_N, _D), jnp.float32),
    )(f, mp, mn, W1, b1, pw_row, W2, b2)


def kernel(feature_a, feature_b, edge_ab_pos, edge_ab_neg, edge_ba_pos, edge_ba_neg,
           W_abp, b_abp, a_abp, W_abn, b_abn, a_abn,
           W_bap, b_bap, a_bap, W_ban, b_ban, a_ban,
           W1, b1, prelu_w, W2, b2):
    fa = jnp.pad(feature_a, ((0, _P - _N), (0, 0)))
    fb = jnp.pad(feature_b, ((0, _P - _N), (0, 0)))
    dsts = [edge_ab_pos, edge_ab_neg, edge_ba_pos, edge_ba_neg]

    params_flat = []
    for W, b, a in ((W_abp, b_abp, a_abp), (W_abn, b_abn, a_abn),
                    (W_bap, b_bap, a_bap), (W_ban, b_ban, a_ban)):
        params_flat += [W, b.reshape(1, _D),
                        a[:_D, 0].reshape(1, _D), a[_D:, 0].reshape(1, _D)]

    res = _embed_call(fa, fb, params_flat)
    embs = res[0:4]
    ssrcs = res[4:8]
    sdsts = res[8:12]

    m0, m1, m2, m3 = _agg_sc(embs, dsts, ssrcs, sdsts)

    j = jnp.arange(_D)
    blk, r = j // 32, j % 32
    src = 32 * blk + jnp.where(r < 16, 2 * r, 2 * (r - 16) + 1)
    W1p = jnp.concatenate([W1[0:_D], W1[_D:2 * _D][src], W1[2 * _D:3 * _D][src]],
                          axis=0)
    b1r = b1.reshape(1, 2 * _D)
    b2r = b2.reshape(1, _D)
    pw_row = jnp.broadcast_to(prelu_w.reshape(1, 1), (1, 2 * _D))
    out_a = _mlp_call(fa, m0, m1, W1p, b1r, pw_row, W2, b2r)
    out_b = _mlp_call(fb, m2, m3, W1p, b1r, pw_row, W2, b2r)
    return (out_a, out_b)


# speedup vs baseline: 21.4169x; 1.0011x over previous
import jax
import jax.numpy as jnp
from jax import lax
from jax.experimental import pallas as pl
from jax.experimental.pallas import tpu as pltpu
from jax.experimental.pallas import tpu_sc as plsc

_D = 128
_N = 10000
_DEG = 32
_P = 10240
_BLK = 512
_GRID = _P // _BLK
_NW = 32
_NPT = 320
_CN = 4
_CE = _CN * _DEG
_NCH = _NPT // _CN


def _embed_body(fa_ref, fb_ref, *refs):
    params = [refs[4 * r: 4 * r + 4] for r in range(4)]
    embs = refs[16:20]
    ssrcs = refs[20:24]
    sdsts = refs[24:28]
    fa = fa_ref[...]
    fb = fb_ref[...]
    dn = (((1,), (1,)), ((), ()))
    for r in range(4):
        W, b, lo, hi = params[r]
        fe = fb if r < 2 else fa
        fs = fa if r < 2 else fb
        emb = jnp.dot(fe, W[...], preferred_element_type=jnp.float32) + b[...]
        embs[r][...] = emb.astype(jnp.bfloat16)
        sd = lax.dot_general(hi[...], emb, dn, preferred_element_type=jnp.float32)
        sdsts[r][...] = sd.reshape(_BLK)
        ss = lax.dot_general(lo[...], fs, dn, preferred_element_type=jnp.float32)
        ssrcs[r][...] = ss.reshape(_BLK)


def _embed_call(fa, fb, params_flat):
    full = lambda i: (0, 0)
    row = lambda i: (i, 0)
    in_specs = [pl.BlockSpec((_BLK, _D), row), pl.BlockSpec((_BLK, _D), row)]
    for _ in range(4):
        in_specs += [
            pl.BlockSpec((_D, _D), full),
            pl.BlockSpec((1, _D), full),
            pl.BlockSpec((1, _D), full),
            pl.BlockSpec((1, _D), full),
        ]
    out_specs = ([pl.BlockSpec((_BLK, _D), row)] * 4
                 + [pl.BlockSpec((_BLK,), lambda i: (i,))] * 8)
    out_shape = ([jax.ShapeDtypeStruct((_P, _D), jnp.bfloat16)] * 4
                 + [jax.ShapeDtypeStruct((_P,), jnp.float32)] * 8)
    return pl.pallas_call(
        _embed_body,
        grid=(_GRID,),
        in_specs=in_specs,
        out_specs=out_specs,
        out_shape=out_shape,
    )(fa, fb, *params_flat)


def _rel_agg(eh, dh, sh, dsh, oh, nbase, ebase, sid,
             idx_v, sdst_v, ssrc_v, out_v, rows0, rows1, w_v, tbl_sp,
             sem0, sem1):
    @pl.when(sid == 0)
    def _():
        pltpu.sync_copy(eh, tbl_sp)
    pltpu.sync_copy(dh.at[1, pl.ds(ebase, _NPT * _DEG)],
                    idx_v.at[pl.ds(0, _NPT * _DEG)])
    pltpu.sync_copy(dh.at[1, pl.ds(ebase, 2 * _CE)],
                    idx_v.at[pl.ds(_NPT * _DEG, 2 * _CE)])
    pltpu.sync_copy(dsh, sdst_v)
    pltpu.sync_copy(sh.at[pl.ds(nbase, _NPT)], ssrc_v.at[pl.ds(0, _NPT)])
    plsc.subcore_barrier()

    def compute(c, rows_v):
        sv = ssrc_v[pl.ds(c * _CN, 16)]
        for n in range(_CN):
            node = c * _CN + n
            eb = n * _DEG
            ssv = sv[n]
            wvecs = []
            tot = jnp.zeros((16,), jnp.float32)
            for k in range(_DEG // 16):
                ii = idx_v[pl.ds(c * _CE + eb + k * 16, 16)]
                sdv = plsc.load_gather(sdst_v, [ii])
                x = ssv + sdv
                w = jnp.exp(jnp.where(x > 0.0, x, 0.1 * (jnp.exp(x) - 1.0)))
                wvecs.append(w)
                tot = tot + w
            rsv = jnp.full((16,), jnp.sum(tot), jnp.float32)
            invv = 1.0 / jnp.where(rsv == 0.0, 1.0, rsv)
            for k in range(_DEG // 16):
                w_v[pl.ds(k * 16, 16)] = wvecs[k] * invv

            def edge(j, acc):
                wj = plsc.load_gather(w_v, [jnp.full((16,), j, jnp.int32)])
                outs = []
                for v in range(_D // 32):
                    pk = rows_v[eb + j, pl.ds(v * 32, 32)]
                    pi = plsc.bitcast(pk, jnp.int32)
                    ev = plsc.bitcast(lax.shift_left(pi, 16), jnp.float32)
                    od = plsc.bitcast(jnp.bitwise_and(pi, jnp.int32(-65536)),
                                      jnp.float32)
                    outs.append(acc[2 * v] + wj * ev)
                    outs.append(acc[2 * v + 1] + wj * od)
                return tuple(outs)

            acc0 = tuple(jnp.zeros((16,), jnp.float32) for _ in range(_D // 16))
            acc = lax.fori_loop(0, _DEG, edge, acc0, unroll=4)
            for v in range(_D // 32):
                out_v[node, pl.ds(v * 32, 16)] = acc[2 * v]
                out_v[node, pl.ds(v * 32 + 16, 16)] = acc[2 * v + 1]

    def irow(c):
        return idx_v.at[pl.ds(c * _CE, _CE)]

    pltpu.async_copy(tbl_sp.at[irow(0)], rows0, sem0)
    pltpu.async_copy(tbl_sp.at[irow(1)], rows1, sem1)

    def pair(g, carry):
        c = 2 * g
        pltpu.make_async_copy(tbl_sp.at[irow(c)], rows0, sem0).wait()
        compute(c, rows0)
        pltpu.async_copy(tbl_sp.at[irow(c + 2)], rows0, sem0)
        pltpu.make_async_copy(tbl_sp.at[irow(c + 1)], rows1, sem1).wait()
        compute(c + 1, rows1)
        pltpu.async_copy(tbl_sp.at[irow(c + 3)], rows1, sem1)
        return carry

    lax.fori_loop(0, _NCH // 2, pair, 0)
    pltpu.make_async_copy(tbl_sp.at[irow(_NCH)], rows0, sem0).wait()
    pltpu.make_async_copy(tbl_sp.at[irow(_NCH + 1)], rows1, sem1).wait()
    pltpu.sync_copy(out_v, oh.at[pl.ds(nbase, _NPT)])
    plsc.subcore_barrier()


def _agg_body(e0, e1, e2, e3, d0, d1, d2, d3,
              ss0, ss1, ss2, ss3, sd0, sd1, sd2, sd3,
              o0, o1, o2, o3,
              idx_v, sdst_v, ssrc_v, out_v, rows0, rows1, w_v, tbl_sp,
              sem0, sem1):
    sid = lax.axis_index("s")
    wid = sid * 2 + lax.axis_index("c")
    nbase = jnp.minimum(wid * _NPT, _N - _NPT)
    ebase = nbase * _DEG
    for eh, dh, sh, dsh, oh in ((e0, d0, ss0, sd0, o0),
                                (e1, d1, ss1, sd1, o1),
                                (e2, d2, ss2, sd2, o2),
                                (e3, d3, ss3, sd3, o3)):
        _rel_agg(eh, dh, sh, dsh, oh, nbase, ebase, sid,
                 idx_v, sdst_v, ssrc_v, out_v, rows0, rows1, w_v, tbl_sp,
                 sem0, sem1)


def _agg_sc(embs, dsts, ssrcs, sdsts):
    mesh = plsc.VectorSubcoreMesh(core_axis_name="c", subcore_axis_name="s")
    call = pl.kernel(
        _agg_body,
        mesh=mesh,
        compiler_params=pltpu.CompilerParams(use_tc_tiling_on_sc=False,
                                             needs_layout_passes=False),
        out_type=[jax.ShapeDtypeStruct((_N, _D), jnp.float32)] * 4,
        scratch_types=[
            pltpu.VMEM((_NPT * _DEG + 2 * _CE,), jnp.int32),
            pltpu.VMEM((_P,), jnp.float32),
            pltpu.VMEM((_NPT + 16,), jnp.float32),
            pltpu.VMEM((_NPT, _D), jnp.float32),
            pltpu.VMEM((_CE, _D), jnp.bfloat16),
            pltpu.VMEM((_CE, _D), jnp.bfloat16),
            pltpu.VMEM((_DEG,), jnp.float32),
            pltpu.VMEM_SHARED((_P, _D), jnp.bfloat16),
            pltpu.SemaphoreType.DMA,
            pltpu.SemaphoreType.DMA,
        ],
    )
    return call(*embs, *dsts, *ssrcs, *sdsts)


def _mlp_body(f_ref, mp_ref, mn_ref, W1_ref, b1_ref, pw_ref, W2_ref, b2_ref, o_ref):
    W1 = W1_ref[...]
    h = (jnp.dot(f_ref[...], W1[0:_D], preferred_element_type=jnp.float32)
         + jnp.dot(mp_ref[...], W1[_D:2 * _D], preferred_element_type=jnp.float32)
         + jnp.dot(mn_ref[...], W1[2 * _D:3 * _D], preferred_element_type=jnp.float32)
         + b1_ref[...])
    h = jnp.where(h >= 0.0, h, pw_ref[...] * h)
    o_ref[...] = jnp.dot(h, W2_ref[...], preferred_element_type=jnp.float32) + b2_ref[...]


def _mlp_call(f, mp, mn, W1, b1, pw_row, W2, b2):
    full = lambda i: (0, 0)
    row = lambda i: (i, 0)
    return pl.pallas_call(
        _mlp_body,
        grid=(_GRID,),
        in_specs=[
            pl.BlockSpec((_BLK, _D), row),
            pl.BlockSpec((_BLK, _D), row),
            pl.BlockSpec((_BLK, _D), row),
            pl.BlockSpec((3 * _D, 2 * _D), full),
            pl.BlockSpec((1, 2 * _D), full),
            pl.BlockSpec((1, 2 * _D), full),
            pl.BlockSpec((2 * _D, _D), full),
            pl.BlockSpec((1, _D), full),
        ],
        out_specs=pl.BlockSpec((_BLK, _D), row),
        out_shape=jax.ShapeDtypeStruct((_N, _D), jnp.float32),
    )(f, mp, mn, W1, b1, pw_row, W2, b2)


def kernel(feature_a, feature_b, edge_ab_pos, edge_ab_neg, edge_ba_pos, edge_ba_neg,
           W_abp, b_abp, a_abp, W_abn, b_abn, a_abn,
           W_bap, b_bap, a_bap, W_ban, b_ban, a_ban,
           W1, b1, prelu_w, W2, b2):
    fa = jnp.pad(feature_a, ((0, _P - _N), (0, 0)))
    fb = jnp.pad(feature_b, ((0, _P - _N), (0, 0)))
    dsts = [edge_ab_pos, edge_ab_neg, edge_ba_pos, edge_ba_neg]

    params_flat = []
    for W, b, a in ((W_abp, b_abp, a_abp), (W_abn, b_abn, a_abn),
                    (W_bap, b_bap, a_bap), (W_ban, b_ban, a_ban)):
        params_flat += [W, b.reshape(1, _D),
                        a[:_D, 0].reshape(1, _D), a[_D:, 0].reshape(1, _D)]

    res = _embed_call(fa, fb, params_flat)
    embs = res[0:4]
    ssrcs = res[4:8]
    sdsts = res[8:12]

    m0, m1, m2, m3 = _agg_sc(embs, dsts, ssrcs, sdsts)

    j = jnp.arange(_D)
    blk, r = j // 32, j % 32
    src = 32 * blk + jnp.where(r < 16, 2 * r, 2 * (r - 16) + 1)
    W1p = jnp.concatenate([W1[0:_D], W1[_D:2 * _D][src], W1[2 * _D:3 * _D][src]],
                          axis=0)
    b1r = b1.reshape(1, 2 * _D)
    b2r = b2.reshape(1, _D)
    pw_row = jnp.broadcast_to(prelu_w.reshape(1, 1), (1, 2 * _D))
    out_a = _mlp_call(fa, m0, m1, W1p, b1r, pw_row, W2, b2r)
    out_b = _mlp_call(fb, m2, m3, W1p, b1r, pw_row, W2, b2r)
    return (out_a, out_b)
